# Initial kernel scaffold; baseline (speedup 1.0000x reference)
import jax
import jax.numpy as jnp
from jax.experimental import pallas as pl


def kernel(x, router_W, router_b, W1, b1, W2, b2):
    raise NotImplementedError("write your pallas kernel here")


# speedup vs baseline: 1.3641x; 1.3641x over previous
import jax
import jax.numpy as jnp
from jax.experimental import pallas as pl
from jax.experimental.pallas import tpu as pltpu

E = 8
TB = 1024


def _moe_body(x_ref, rw_ref, rb_ref, w1_ref, b1_ref, w2_ref, b2_ref,
              out_ref, wts_ref):
    e = pl.program_id(1)

    @pl.when(e == 0)
    def _():
        logits = jax.lax.dot_general(
            x_ref[...], rw_ref[...], (((1,), (1,)), ((), ())),
            preferred_element_type=jnp.float32) + rb_ref[...][None, :]
        iota = jax.lax.broadcasted_iota(jnp.int32, logits.shape, 1)
        v1 = jnp.max(logits, axis=1, keepdims=True)
        a1 = jnp.min(jnp.where(logits == v1, iota, E), axis=1, keepdims=True)
        masked = jnp.where(iota == a1, -jnp.inf, logits)
        v2 = jnp.max(masked, axis=1, keepdims=True)
        a2 = jnp.min(jnp.where(masked == v2, iota, E), axis=1, keepdims=True)
        ez = jnp.exp(v2 - v1)
        g1 = 1.0 / (1.0 + ez)
        g2 = 1.0 - g1
        wts_ref[...] = (jnp.where(iota == a1, g1, 0.0)
                        + jnp.where(iota == a2, g2, 0.0))
        out_ref[...] = jnp.zeros_like(out_ref)

    h = jax.lax.dot_general(x_ref[...], w1_ref[0],
                            (((1,), (1,)), ((), ())),
                            preferred_element_type=jnp.float32)
    h = jnp.maximum(h + b1_ref[0], 0.0)
    o = jax.lax.dot_general(h, w2_ref[0],
                            (((1,), (1,)), ((), ())),
                            preferred_element_type=jnp.float32)
    o = o + b2_ref[0]
    wts = wts_ref[...]
    lane = jax.lax.broadcasted_iota(jnp.int32, wts.shape, 1)
    wsel = jnp.sum(jnp.where(lane == e, wts, 0.0), axis=1, keepdims=True)
    out_ref[...] += wsel * o


def kernel(x, router_W, router_b, W1, b1, W2, b2):
    bsz, slen, dim = x.shape
    x_flat = x.reshape(bsz * slen, dim)
    T = x_flat.shape[0]
    H = W1.shape[1]
    nt = T // TB

    out = pl.pallas_call(
        _moe_body,
        grid=(nt, E),
        in_specs=[
            pl.BlockSpec((TB, dim), lambda i, e: (i, 0)),
            pl.BlockSpec((E, dim), lambda i, e: (0, 0)),
            pl.BlockSpec((E,), lambda i, e: (0,)),
            pl.BlockSpec((1, H, dim), lambda i, e: (e, 0, 0)),
            pl.BlockSpec((1, 1, H), lambda i, e: (e, 0, 0)),
            pl.BlockSpec((1, dim, H), lambda i, e: (e, 0, 0)),
            pl.BlockSpec((1, 1, dim), lambda i, e: (e, 0, 0)),
        ],
        out_specs=pl.BlockSpec((TB, dim), lambda i, e: (i, 0)),
        out_shape=jax.ShapeDtypeStruct((T, dim), jnp.float32),
        scratch_shapes=[pltpu.VMEM((TB, E), jnp.float32)],
    )(x_flat, router_W, router_b, W1, b1.reshape(E, 1, H),
      W2, b2.reshape(E, 1, dim))
    return out.reshape(bsz, slen, dim)
